# baseline (speedup 1.0000x reference)
import functools

import jax
import jax.numpy as jnp
from jax import lax
from jax.experimental import pallas as pl
from jax.experimental.pallas import tpu as pltpu
from jax.experimental.pallas import tpu_sc as plsc

BATCH = 16384
F = 4096
L = 16
NW = 32
ROWS_PER_TILE = BATCH // NW
CHUNK = 8
NCHUNKS = ROWS_PER_TILE // CHUNK
JBLOCKS = F // L


def _permute_body(in_hbm, perm_hbm, out_hbm, perm_v, in_v, out_v):
    wid = lax.axis_index("s") * 2 + lax.axis_index("c")
    base = wid * ROWS_PER_TILE
    pltpu.sync_copy(perm_hbm, perm_v)

    def chunk_body(c, _):
        e0 = (base + c * CHUNK) * F
        pltpu.sync_copy(in_hbm.at[pl.ds(e0, CHUNK * F)], in_v)

        def j_body(j, _):
            col0 = pl.multiple_of(j * L, L)
            idx = perm_v[pl.ds(col0, L)]
            for r in range(CHUNK):
                vals = plsc.load_gather(in_v, [idx + r * F])
                out_v[pl.ds(r * F + col0, L)] = vals
            return 0

        lax.fori_loop(0, JBLOCKS, j_body, 0)
        pltpu.sync_copy(out_v, out_hbm.at[pl.ds(e0, CHUNK * F)])
        return 0

    lax.fori_loop(0, NCHUNKS, chunk_body, 0)


@functools.partial(
    pl.kernel,
    mesh=plsc.VectorSubcoreMesh(core_axis_name="c", subcore_axis_name="s"),
    out_type=jax.ShapeDtypeStruct((BATCH * F,), jnp.float32),
    scratch_types=[
        pltpu.VMEM((F,), jnp.int32),
        pltpu.VMEM((CHUNK * F,), jnp.float32),
        pltpu.VMEM((CHUNK * F,), jnp.float32),
    ],
    compiler_params=pltpu.CompilerParams(needs_layout_passes=False),
)
def _permute_kernel(in_hbm, perm_hbm, out_hbm, perm_v, in_v, out_v):
    _permute_body(in_hbm, perm_hbm, out_hbm, perm_v, in_v, out_v)


def kernel(inputs, permutation):
    flat = _permute_kernel(inputs.reshape(-1), permutation.astype(jnp.int32))
    outputs = flat.reshape(BATCH, F)
    logabsdet = jnp.zeros((inputs.shape[0],), dtype=inputs.dtype)
    return (outputs, logabsdet)


# speedup vs baseline: 1.4751x; 1.4751x over previous
import functools

import jax
import jax.numpy as jnp
from jax import lax
from jax.experimental import pallas as pl
from jax.experimental.pallas import tpu as pltpu
from jax.experimental.pallas import tpu_sc as plsc

BATCH = 16384
F = 4096
L = 16
NW = 32
ROWS_PER_TILE = BATCH // NW
CHUNK = 8
NCHUNKS = ROWS_PER_TILE // CHUNK
JBLOCKS = F // L


def _permute_body(in_hbm, perm_hbm, out_hbm, perm_v, in_v, out_v):
    wid = lax.axis_index("s") * 2 + lax.axis_index("c")
    base = wid * ROWS_PER_TILE
    pltpu.sync_copy(perm_hbm, perm_v)

    def chunk_body(c, _):
        r0 = base + c * CHUNK
        pltpu.sync_copy(in_hbm.at[pl.ds(r0, CHUNK)], in_v)

        def j_body(j, _):
            col0 = pl.multiple_of(j * L, L)
            idx = perm_v[pl.ds(col0, L)]
            for r in range(CHUNK):
                row_idx = jnp.full((L,), r, dtype=jnp.int32)
                vals = plsc.load_gather(in_v, [row_idx, idx])
                out_v[r, pl.ds(col0, L)] = vals
            return 0

        lax.fori_loop(0, JBLOCKS, j_body, 0)
        pltpu.sync_copy(out_v, out_hbm.at[pl.ds(r0, CHUNK)])
        return 0

    lax.fori_loop(0, NCHUNKS, chunk_body, 0)


@functools.partial(
    pl.kernel,
    mesh=plsc.VectorSubcoreMesh(core_axis_name="c", subcore_axis_name="s"),
    out_type=jax.ShapeDtypeStruct((BATCH, F), jnp.float32),
    scratch_types=[
        pltpu.VMEM((F,), jnp.int32),
        pltpu.VMEM((CHUNK, F), jnp.float32),
        pltpu.VMEM((CHUNK, F), jnp.float32),
    ],
    compiler_params=pltpu.CompilerParams(needs_layout_passes=False),
)
def _permute_kernel(in_hbm, perm_hbm, out_hbm, perm_v, in_v, out_v):
    _permute_body(in_hbm, perm_hbm, out_hbm, perm_v, in_v, out_v)


def kernel(inputs, permutation):
    outputs = _permute_kernel(inputs, permutation.astype(jnp.int32))
    logabsdet = jnp.zeros((inputs.shape[0],), dtype=inputs.dtype)
    return (outputs, logabsdet)
